# baseline (speedup 1.0000x reference)
import jax
import jax.numpy as jnp
from jax.experimental import pallas as pl
from jax.experimental.pallas import tpu as pltpu

_U = 4
_S = 3072
_BM = _U * _S


def _matmul_stage(k, x_ref, ct2, b2, scr):
    x = x_ref[k * _S:(k + 1) * _S, :]
    dots2 = jax.lax.dot_general(
        x, ct2, (((1,), (0,)), ((), ())),
        preferred_element_type=jnp.float32,
        precision=jax.lax.Precision.DEFAULT,
    )
    scr[...] = b2 + dots2


def _label_stage(k, iwt, scr, out_ref):
    d2 = scr[...]
    rowmin = jnp.min(d2, axis=1, keepdims=True)
    maskf = jnp.where(d2 <= rowmin, 1.0, 0.0)
    sums = jax.lax.dot_general(
        iwt, maskf, (((1,), (1,)), ((), ())),
        preferred_element_type=jnp.float32,
        precision=jax.lax.Precision.DEFAULT,
    )
    idx = (8.0 * sums[0:1, :] + sums[1:2, :]) / sums[2:3, :]
    out_ref[k * _S:(k + 1) * _S] = idx.reshape(-1).astype(jnp.int32)


def _labeller_body(x_ref, ct2_ref, b2_ref, iwt_ref, out_ref, *scrs):
    ct2 = ct2_ref[...]
    b2 = b2_ref[...]
    iwt = iwt_ref[...]
    _matmul_stage(0, x_ref, ct2, b2, scrs[0])
    for k in range(1, _U):
        _matmul_stage(k, x_ref, ct2, b2, scrs[k])
        _label_stage(k - 1, iwt, scrs[k - 1], out_ref)
    _label_stage(_U - 1, iwt, scrs[_U - 1], out_ref)


def kernel(inpt, centers0, centers1):
    B, T, C = inpt.shape
    M = B * T
    x2d = inpt.reshape(M, C)
    ct2 = centers1.T * -2.0
    b2 = jnp.sum(centers1 * centers1, axis=1)[None]
    j = jnp.arange(1024, dtype=jnp.float32)
    iwt = jnp.stack(
        [jnp.floor(j / 8.0), jnp.mod(j, 8.0), jnp.ones_like(j)]
        + [jnp.zeros_like(j)] * 5,
        axis=0,
    )
    out = pl.pallas_call(
        _labeller_body,
        grid=(M // _BM,),
        in_specs=[
            pl.BlockSpec((_BM, 128), lambda i: (i, 1)),
            pl.BlockSpec((128, 1024), lambda i: (0, 0)),
            pl.BlockSpec((1, 1024), lambda i: (0, 0)),
            pl.BlockSpec((8, 1024), lambda i: (0, 0)),
        ],
        out_specs=pl.BlockSpec((_BM,), lambda i: (i,)),
        out_shape=jax.ShapeDtypeStruct((M,), jnp.int32),
        scratch_shapes=[pltpu.VMEM((_S, 1024), jnp.float32) for _ in range(_U)],
    )(x2d, ct2, b2, iwt)
    return out.reshape(B, T)


# speedup vs baseline: 1.0209x; 1.0209x over previous
import jax
import jax.numpy as jnp
from jax.experimental import pallas as pl
from jax.experimental.pallas import tpu as pltpu

_U = 1
_S = 4096
_BM = _U * _S


def _matmul_stage(k, x_ref, ct2, b2, scr):
    x = x_ref[k * _S:(k + 1) * _S, :]
    dots2 = jax.lax.dot_general(
        x, ct2, (((1,), (0,)), ((), ())),
        preferred_element_type=jnp.float32,
        precision=jax.lax.Precision.DEFAULT,
    )
    scr[...] = b2 + dots2


def _label_stage(k, iwt, scr, out_ref):
    d2 = scr[...]
    rowmin = jnp.min(d2, axis=1, keepdims=True)
    maskf = jnp.where(d2 <= rowmin, 1.0, 0.0)
    sums = jax.lax.dot_general(
        iwt, maskf, (((1,), (1,)), ((), ())),
        preferred_element_type=jnp.float32,
        precision=jax.lax.Precision.DEFAULT,
    )
    idx = (8.0 * sums[0:1, :] + sums[1:2, :]) / sums[2:3, :]
    out_ref[k * _S:(k + 1) * _S] = idx.reshape(-1).astype(jnp.int32)


def _labeller_body(x_ref, ct2_ref, b2_ref, iwt_ref, out_ref, *scrs):
    ct2 = ct2_ref[...]
    b2 = b2_ref[...]
    iwt = iwt_ref[...]
    _matmul_stage(0, x_ref, ct2, b2, scrs[0])
    for k in range(1, _U):
        _matmul_stage(k, x_ref, ct2, b2, scrs[k])
        _label_stage(k - 1, iwt, scrs[k - 1], out_ref)
    _label_stage(_U - 1, iwt, scrs[_U - 1], out_ref)


def kernel(inpt, centers0, centers1):
    B, T, C = inpt.shape
    M = B * T
    x2d = inpt.reshape(M, C)
    ct2 = centers1.T * -2.0
    b2 = jnp.sum(centers1 * centers1, axis=1)[None]
    j = jnp.arange(1024, dtype=jnp.float32)
    iwt = jnp.stack(
        [jnp.floor(j / 8.0), jnp.mod(j, 8.0), jnp.ones_like(j)]
        + [jnp.zeros_like(j)] * 5,
        axis=0,
    )
    out = pl.pallas_call(
        _labeller_body,
        grid=(M // _BM,),
        in_specs=[
            pl.BlockSpec((_BM, 128), lambda i: (i, 1)),
            pl.BlockSpec((128, 1024), lambda i: (0, 0)),
            pl.BlockSpec((1, 1024), lambda i: (0, 0)),
            pl.BlockSpec((8, 1024), lambda i: (0, 0)),
        ],
        out_specs=pl.BlockSpec((_BM,), lambda i: (i,)),
        out_shape=jax.ShapeDtypeStruct((M,), jnp.int32),
        scratch_shapes=[pltpu.VMEM((_S, 1024), jnp.float32) for _ in range(_U)],
    )(x2d, ct2, b2, iwt)
    return out.reshape(B, T)


# speedup vs baseline: 1.0231x; 1.0021x over previous
import jax
import jax.numpy as jnp
from jax.experimental import pallas as pl

_BM = 6144


def _labeller_body(x_ref, ct2_ref, b2_ref, iwt_ref, out_ref):
    x = x_ref[...]
    ct2 = ct2_ref[...]
    dots2 = jax.lax.dot_general(
        x, ct2, (((1,), (0,)), ((), ())),
        preferred_element_type=jnp.float32,
        precision=jax.lax.Precision.DEFAULT,
    )
    d2 = b2_ref[...] + dots2
    rowmin = jnp.min(d2, axis=1, keepdims=True)
    maskf = jnp.where(d2 <= rowmin, 1.0, 0.0)
    sums = jax.lax.dot_general(
        iwt_ref[...], maskf, (((1,), (1,)), ((), ())),
        preferred_element_type=jnp.float32,
        precision=jax.lax.Precision.DEFAULT,
    )
    idx = (8.0 * sums[0:1, :] + sums[1:2, :]) / sums[2:3, :]
    out_ref[...] = idx.reshape(-1).astype(jnp.int32)


def kernel(inpt, centers0, centers1):
    B, T, C = inpt.shape
    M = B * T
    x2d = inpt.reshape(M, C)
    ct2 = centers1.T * -2.0
    b2 = jnp.sum(centers1 * centers1, axis=1)[None]
    j = jnp.arange(1024, dtype=jnp.float32)
    iwt = jnp.stack(
        [jnp.floor(j / 8.0), jnp.mod(j, 8.0), jnp.ones_like(j)]
        + [jnp.zeros_like(j)] * 5,
        axis=0,
    )
    out = pl.pallas_call(
        _labeller_body,
        grid=(M // _BM,),
        in_specs=[
            pl.BlockSpec((_BM, 128), lambda i: (i, 1)),
            pl.BlockSpec((128, 1024), lambda i: (0, 0)),
            pl.BlockSpec((1, 1024), lambda i: (0, 0)),
            pl.BlockSpec((8, 1024), lambda i: (0, 0)),
        ],
        out_specs=pl.BlockSpec((_BM,), lambda i: (i,)),
        out_shape=jax.ShapeDtypeStruct((M,), jnp.int32),
    )(x2d, ct2, b2, iwt)
    return out.reshape(B, T)
